# Initial kernel scaffold; baseline (speedup 1.0000x reference)
import jax
import jax.numpy as jnp
from jax.experimental import pallas as pl


def kernel(source, target, cost_matrix):
    raise NotImplementedError("write your pallas kernel here")


# speedup vs baseline: 44.4110x; 44.4110x over previous
import functools

import jax
import jax.numpy as jnp
from jax import lax
from jax.experimental import pallas as pl
from jax.experimental.pallas import tpu as pltpu
from jax.experimental.pallas import tpu_sc as plsc

B = 8
NS = 2048
N = NS * NS
K_SEL = int(N * 0.2)
NBINS = 2048
LANES = 16
TPB = 4
SHARD = N // TPB
CHUNK = 8192
NCHUNK = SHARD // CHUNK
VPC = CHUNK // LANES
UNROLL = 4
ROWS = NBINS // LANES

_mesh = plsc.VectorSubcoreMesh(core_axis_name="c", subcore_axis_name="s")


def _scan_threshold(h, rank):
  zeros = jnp.zeros((LANES,), jnp.int32)

  def body(j, carry):
    cum_prev, cnt, maxlt = carry
    v = h[j]
    cs = plsc.cumsum(v) + cum_prev
    lt = cs < rank
    cnt = cnt + lt.astype(jnp.int32)
    maxlt = jnp.where(lt, cs, maxlt)
    return cum_prev + jnp.sum(v), cnt, maxlt

  _, cnt, maxlt = lax.fori_loop(0, ROWS, body, (jnp.int32(0), zeros, zeros))
  return jnp.sum(cnt), jnp.max(maxlt)


def _sc_body(cost_hbm, out_hbm, folds_hbm, hist, fold, hmrg,
             buf0, buf1, obuf0, obuf1, isem0, isem1, osem0, osem1):
  cid = lax.axis_index("c")
  sid = lax.axis_index("s")
  bl = sid // 4
  q = sid % 4
  batch = cid * 4 + bl
  base = q * SHARD
  lane = lax.iota(jnp.int32, LANES)
  ones = jnp.full((LANES,), 1, jnp.int32)
  zeros = jnp.zeros((LANES,), jnp.int32)
  zerosf = jnp.zeros((LANES,), jnp.float32)

  ibufs = (buf0, buf1)
  isems = (isem0, isem1)
  obufs = (obuf0, obuf1)
  osems = (osem0, osem1)

  def in_copy(ci, b):
    return pltpu.make_async_copy(
        cost_hbm.at[batch, pl.ds(base + ci * CHUNK, CHUNK)],
        ibufs[b], isems[b])

  def out_copy(ci, b):
    return pltpu.make_async_copy(
        obufs[b],
        out_hbm.at[batch, pl.ds(base + ci * CHUNK, CHUNK)], osems[b])

  def zero_hist(_i, _):
    for u in range(8):
      hist[pl.ds((_i * 8 + u) * LANES, LANES)] = zeros
    return _

  def fold_hist(j, _):
    rows16 = (j * LANES + lane) * LANES
    accs = [zeros] * 4
    for l in range(LANES):
      accs[l % 4] = accs[l % 4] + plsc.load_gather(hist, [rows16 + l])
    fold[j] = (accs[0] + accs[1]) + (accs[2] + accs[3])
    return _

  def merge_and_scan(rank):
    lax.fori_loop(0, ROWS, fold_hist, 0)
    pltpu.sync_copy(fold, folds_hbm.at[cid, sid])
    plsc.subcore_barrier()

    def add_rows(j, _):
      fold[j] = fold[j] + hmrg[j]
      return _

    for dt in range(1, TPB):
      other = bl * TPB + lax.rem(q + dt, TPB)
      pltpu.sync_copy(folds_hbm.at[cid, other], hmrg)
      lax.fori_loop(0, ROWS, add_rows, 0)
    plsc.subcore_barrier()
    return _scan_threshold(fold, rank)

  def stream_pass(process):
    in_copy(0, 0).start()

    def pair(g, _):
      ci0 = g * 2
      in_copy(ci0 + 1, 1).start()
      in_copy(ci0, 0).wait()
      process(ibufs[0])

      @pl.when(ci0 + 2 < NCHUNK)
      def _():
        in_copy(ci0 + 2, 0).start()

      in_copy(ci0 + 1, 1).wait()
      process(ibufs[1])
      return 0

    lax.fori_loop(0, NCHUNK // 2, pair, 0)

  lax.fori_loop(0, NBINS * LANES // (8 * LANES), zero_hist, 0)

  def p1(buf):
    def v1(i, _):
      for u in range(UNROLL):
        c = buf[pl.ds((i * UNROLL + u) * LANES, LANES)]
        b1 = (c * 2048.0).astype(jnp.int32)
        plsc.addupdate_scatter(hist, [(b1 * LANES) | lane], ones)
      return _

    lax.fori_loop(0, VPC // UNROLL, v1, 0)

  stream_pass(p1)
  bstar, cum_before = merge_and_scan(K_SEL)
  r = K_SEL - cum_before

  lax.fori_loop(0, NBINS * LANES // (8 * LANES), zero_hist, 0)

  def p2(buf):
    def v2(i, _):
      for u in range(UNROLL):
        c = buf[pl.ds((i * UNROLL + u) * LANES, LANES)]
        m = (c * 4194304.0).astype(jnp.int32)
        inbin = (m >> 11) == bstar
        plsc.addupdate_scatter(hist, [((m & 2047) * LANES) | lane], ones,
                               mask=inbin)
      return _

    lax.fori_loop(0, VPC // UNROLL, v2, 0)

  stream_pass(p2)
  sbstar, _cb2 = merge_and_scan(r)

  vthr = (bstar * NBINS + sbstar + 1).astype(jnp.float32) * (2.0 ** -22)

  in_copy(0, 0).start()

  def pair3(g, _):
    ci0 = g * 2
    in_copy(ci0 + 1, 1).start()

    for b in range(2):
      ci = ci0 + b
      in_copy(ci, b).wait()

      @pl.when(g > 0)
      def _():
        out_copy(ci - 2, b).wait()

      buf, obuf = ibufs[b], obufs[b]

      def v3(i, _):
        for u in range(UNROLL):
          s = pl.ds((i * UNROLL + u) * LANES, LANES)
          c = buf[s]
          obuf[s] = jnp.where(c < vthr, c, zerosf)
        return _

      lax.fori_loop(0, VPC // UNROLL, v3, 0)
      out_copy(ci, b).start()

      if b == 0:
        @pl.when(ci0 + 2 < NCHUNK)
        def _():
          in_copy(ci0 + 2, 0).start()

    return 0

  lax.fori_loop(0, NCHUNK // 2, pair3, 0)
  out_copy(NCHUNK - 2, 0).wait()
  out_copy(NCHUNK - 1, 1).wait()


_select = functools.partial(
    pl.kernel,
    out_type=(
        jax.ShapeDtypeStruct((B, N), jnp.float32),
        jax.ShapeDtypeStruct((2, 16, ROWS, LANES), jnp.int32),
    ),
    mesh=_mesh,
    compiler_params=pltpu.CompilerParams(needs_layout_passes=False),
    scratch_types=[
        pltpu.VMEM((NBINS * LANES,), jnp.int32),
        pltpu.VMEM((ROWS, LANES), jnp.int32),
        pltpu.VMEM((ROWS, LANES), jnp.int32),
        pltpu.VMEM((CHUNK,), jnp.float32),
        pltpu.VMEM((CHUNK,), jnp.float32),
        pltpu.VMEM((CHUNK,), jnp.float32),
        pltpu.VMEM((CHUNK,), jnp.float32),
        pltpu.SemaphoreType.DMA,
        pltpu.SemaphoreType.DMA,
        pltpu.SemaphoreType.DMA,
        pltpu.SemaphoreType.DMA,
    ],
)(_sc_body)


def kernel(source, target, cost_matrix):
  cf = cost_matrix.reshape(B, N)
  sparse, _ = _select(cf)
  return source, target, sparse.reshape(B, NS, NS)


# speedup vs baseline: 47.4613x; 1.0687x over previous
import functools

import jax
import jax.numpy as jnp
from jax import lax
from jax.experimental import pallas as pl
from jax.experimental.pallas import tpu as pltpu
from jax.experimental.pallas import tpu_sc as plsc

B = 8
NS = 2048
N = NS * NS
K_SEL = int(N * 0.2)
NBINS = 2048
LANES = 16
TPB = 4
SHARD = N // TPB
CHUNK = 8192
NCHUNK = SHARD // CHUNK
VPC = CHUNK // LANES
UNROLL = 4
ROWS = NBINS // LANES
RB = 256

_mesh = plsc.VectorSubcoreMesh(core_axis_name="c", subcore_axis_name="s")


def _scan_threshold(h, rank):
  zeros = jnp.zeros((LANES,), jnp.int32)

  def body(j, carry):
    cum_prev, cnt, maxlt = carry
    v = h[j]
    cs = plsc.cumsum(v) + cum_prev
    lt = cs < rank
    cnt = cnt + lt.astype(jnp.int32)
    maxlt = jnp.where(lt, cs, maxlt)
    return cum_prev + jnp.sum(v), cnt, maxlt

  _, cnt, maxlt = lax.fori_loop(0, ROWS, body, (jnp.int32(0), zeros, zeros))
  return jnp.sum(cnt), jnp.max(maxlt)


def _sc_body(cost_hbm, thr_hbm, folds_hbm, hist, fold, hmrg,
             buf0, buf1, isem0, isem1):
  cid = lax.axis_index("c")
  sid = lax.axis_index("s")
  bl = sid // 4
  q = sid % 4
  batch = cid * 4 + bl
  base = q * SHARD
  lane = lax.iota(jnp.int32, LANES)
  ones = jnp.full((LANES,), 1, jnp.int32)
  zeros = jnp.zeros((LANES,), jnp.int32)

  ibufs = (buf0, buf1)
  isems = (isem0, isem1)

  def in_copy(ci, b):
    return pltpu.make_async_copy(
        cost_hbm.at[batch, pl.ds(base + ci * CHUNK, CHUNK)],
        ibufs[b], isems[b])

  def zero_hist(_i, _):
    for u in range(8):
      hist[pl.ds((_i * 8 + u) * LANES, LANES)] = zeros
    return _

  def fold_hist(j, _):
    rows16 = (j * LANES + lane) * LANES
    accs = [zeros] * 4
    for l in range(LANES):
      accs[l % 4] = accs[l % 4] + plsc.load_gather(hist, [rows16 + l])
    fold[j] = (accs[0] + accs[1]) + (accs[2] + accs[3])
    return _

  def merge_and_scan(rank):
    lax.fori_loop(0, ROWS, fold_hist, 0)
    pltpu.sync_copy(fold, folds_hbm.at[cid, sid])
    plsc.subcore_barrier()

    def add_rows(j, _):
      fold[j] = fold[j] + hmrg[j]
      return _

    for dt in range(1, TPB):
      other = bl * TPB + lax.rem(q + dt, TPB)
      pltpu.sync_copy(folds_hbm.at[cid, other], hmrg)
      lax.fori_loop(0, ROWS, add_rows, 0)
    plsc.subcore_barrier()
    return _scan_threshold(fold, rank)

  def stream_pass(process):
    in_copy(0, 0).start()

    def pair(g, _):
      ci0 = g * 2
      in_copy(ci0 + 1, 1).start()
      in_copy(ci0, 0).wait()
      process(ibufs[0])

      @pl.when(ci0 + 2 < NCHUNK)
      def _():
        in_copy(ci0 + 2, 0).start()

      in_copy(ci0 + 1, 1).wait()
      process(ibufs[1])
      return 0

    lax.fori_loop(0, NCHUNK // 2, pair, 0)

  lax.fori_loop(0, NBINS * LANES // (8 * LANES), zero_hist, 0)

  def p1(buf):
    def v1(i, _):
      for u in range(UNROLL):
        c = buf[pl.ds((i * UNROLL + u) * LANES, LANES)]
        b1 = (c * 2048.0).astype(jnp.int32)
        plsc.addupdate_scatter(hist, [(b1 * LANES) | lane], ones)
      return _

    lax.fori_loop(0, VPC // UNROLL, v1, 0)

  stream_pass(p1)
  bstar, cum_before = merge_and_scan(K_SEL)
  r = K_SEL - cum_before

  lax.fori_loop(0, NBINS * LANES // (8 * LANES), zero_hist, 0)

  def p2(buf):
    def v2(i, _):
      for u in range(UNROLL):
        c = buf[pl.ds((i * UNROLL + u) * LANES, LANES)]
        m = (c * 4194304.0).astype(jnp.int32)
        inbin = (m >> 11) == bstar
        plsc.addupdate_scatter(hist, [((m & 2047) * LANES) | lane], ones,
                               mask=inbin)
      return _

    lax.fori_loop(0, VPC // UNROLL, v2, 0)

  stream_pass(p2)
  sbstar, _cb2 = merge_and_scan(r)

  t = bstar * NBINS + sbstar + 1

  @pl.when(q == 0)
  def _():
    fold[0] = zeros + t
    pltpu.sync_copy(fold.at[0], thr_hbm.at[batch])


_select = functools.partial(
    pl.kernel,
    out_type=(
        jax.ShapeDtypeStruct((B, LANES), jnp.int32),
        jax.ShapeDtypeStruct((2, 16, ROWS, LANES), jnp.int32),
    ),
    mesh=_mesh,
    compiler_params=pltpu.CompilerParams(needs_layout_passes=False),
    scratch_types=[
        pltpu.VMEM((NBINS * LANES,), jnp.int32),
        pltpu.VMEM((ROWS, LANES), jnp.int32),
        pltpu.VMEM((ROWS, LANES), jnp.int32),
        pltpu.VMEM((CHUNK,), jnp.float32),
        pltpu.VMEM((CHUNK,), jnp.float32),
        pltpu.SemaphoreType.DMA,
        pltpu.SemaphoreType.DMA,
    ],
)(_sc_body)


def _mask_body(thr_ref, c_ref, o_ref):
  b = pl.program_id(0)
  v = thr_ref[b, 0].astype(jnp.float32) * (2.0 ** -22)
  c = c_ref[...]
  o_ref[...] = jnp.where(c < v, c, 0.0)


_mask = pl.pallas_call(
    _mask_body,
    grid=(B, NS // RB),
    in_specs=[
        pl.BlockSpec(memory_space=pltpu.SMEM),
        pl.BlockSpec((1, RB, NS), lambda b, j: (b, j, 0)),
    ],
    out_specs=pl.BlockSpec((1, RB, NS), lambda b, j: (b, j, 0)),
    out_shape=jax.ShapeDtypeStruct((B, NS, NS), jnp.float32),
)


def kernel(source, target, cost_matrix):
  cf = cost_matrix.reshape(B, N)
  ithr, _ = _select(cf)
  sparse = _mask(ithr, cost_matrix)
  return source, target, sparse


# speedup vs baseline: 47.4827x; 1.0005x over previous
import functools

import jax
import jax.numpy as jnp
from jax import lax
from jax.experimental import pallas as pl
from jax.experimental.pallas import tpu as pltpu
from jax.experimental.pallas import tpu_sc as plsc

B = 8
NS = 2048
N = NS * NS
K_SEL = int(N * 0.2)
NBINS = 2048
LANES = 16
TPB = 4
SHARD = N // TPB
CHUNK = 16384
NCHUNK = SHARD // CHUNK
VPC = CHUNK // LANES
UNROLL = 4
ROWS = NBINS // LANES
RB = 256

_mesh = plsc.VectorSubcoreMesh(core_axis_name="c", subcore_axis_name="s")


def _scan_threshold(h, rank):
  zeros = jnp.zeros((LANES,), jnp.int32)

  def body(j, carry):
    cum_prev, cnt, maxlt = carry
    v = h[j]
    cs = plsc.cumsum(v) + cum_prev
    lt = cs < rank
    cnt = cnt + lt.astype(jnp.int32)
    maxlt = jnp.where(lt, cs, maxlt)
    return cum_prev + jnp.sum(v), cnt, maxlt

  _, cnt, maxlt = lax.fori_loop(0, ROWS, body, (jnp.int32(0), zeros, zeros))
  return jnp.sum(cnt), jnp.max(maxlt)


def _sc_body(cost_hbm, thr_hbm, folds_hbm, hist, fold, hmrg,
             buf0, buf1, isem0, isem1):
  cid = lax.axis_index("c")
  sid = lax.axis_index("s")
  bl = sid // 4
  q = sid % 4
  batch = cid * 4 + bl
  base = q * SHARD
  lane = lax.iota(jnp.int32, LANES)
  ones = jnp.full((LANES,), 1, jnp.int32)
  zeros = jnp.zeros((LANES,), jnp.int32)

  ibufs = (buf0, buf1)
  isems = (isem0, isem1)

  def in_copy(ci, b):
    return pltpu.make_async_copy(
        cost_hbm.at[batch, pl.ds(base + ci * CHUNK, CHUNK)],
        ibufs[b], isems[b])

  def zero_hist(_i, _):
    for u in range(8):
      hist[pl.ds((_i * 8 + u) * LANES, LANES)] = zeros
    return _

  def fold_hist(j, _):
    rows16 = (j * LANES + lane) * LANES
    accs = [zeros] * 4
    for l in range(LANES):
      accs[l % 4] = accs[l % 4] + plsc.load_gather(hist, [rows16 + l])
    fold[j] = (accs[0] + accs[1]) + (accs[2] + accs[3])
    return _

  def merge_and_scan(rank):
    lax.fori_loop(0, ROWS, fold_hist, 0)
    pltpu.sync_copy(fold, folds_hbm.at[cid, sid])
    plsc.subcore_barrier()

    def add_rows(j, _):
      fold[j] = fold[j] + hmrg[j]
      return _

    for dt in range(1, TPB):
      other = bl * TPB + lax.rem(q + dt, TPB)
      pltpu.sync_copy(folds_hbm.at[cid, other], hmrg)
      lax.fori_loop(0, ROWS, add_rows, 0)
    plsc.subcore_barrier()
    return _scan_threshold(fold, rank)

  def stream_pass(process):
    in_copy(0, 0).start()

    def pair(g, _):
      ci0 = g * 2
      in_copy(ci0 + 1, 1).start()
      in_copy(ci0, 0).wait()
      process(ibufs[0])

      @pl.when(ci0 + 2 < NCHUNK)
      def _():
        in_copy(ci0 + 2, 0).start()

      in_copy(ci0 + 1, 1).wait()
      process(ibufs[1])
      return 0

    lax.fori_loop(0, NCHUNK // 2, pair, 0)

  lax.fori_loop(0, NBINS * LANES // (8 * LANES), zero_hist, 0)

  def p1(buf):
    def v1(i, _):
      for u in range(UNROLL):
        c = buf[pl.ds((i * UNROLL + u) * LANES, LANES)]
        b1 = (c * 2048.0).astype(jnp.int32)
        plsc.addupdate_scatter(hist, [(b1 * LANES) | lane], ones)
      return _

    lax.fori_loop(0, VPC // UNROLL, v1, 0)

  stream_pass(p1)
  bstar, cum_before = merge_and_scan(K_SEL)
  r = K_SEL - cum_before

  lax.fori_loop(0, NBINS * LANES // (8 * LANES), zero_hist, 0)

  def p2(buf):
    def v2(i, _):
      for u in range(UNROLL):
        c = buf[pl.ds((i * UNROLL + u) * LANES, LANES)]
        m = (c * 4194304.0).astype(jnp.int32)
        inbin = (m >> 11) == bstar
        plsc.addupdate_scatter(hist, [((m & 2047) * LANES) | lane], ones,
                               mask=inbin)
      return _

    lax.fori_loop(0, VPC // UNROLL, v2, 0)

  stream_pass(p2)
  sbstar, _cb2 = merge_and_scan(r)

  t = bstar * NBINS + sbstar + 1

  @pl.when(q == 0)
  def _():
    fold[0] = zeros + t
    pltpu.sync_copy(fold.at[0], thr_hbm.at[batch])


_select = functools.partial(
    pl.kernel,
    out_type=(
        jax.ShapeDtypeStruct((B, LANES), jnp.int32),
        jax.ShapeDtypeStruct((2, 16, ROWS, LANES), jnp.int32),
    ),
    mesh=_mesh,
    compiler_params=pltpu.CompilerParams(needs_layout_passes=False),
    scratch_types=[
        pltpu.VMEM((NBINS * LANES,), jnp.int32),
        pltpu.VMEM((ROWS, LANES), jnp.int32),
        pltpu.VMEM((ROWS, LANES), jnp.int32),
        pltpu.VMEM((CHUNK,), jnp.float32),
        pltpu.VMEM((CHUNK,), jnp.float32),
        pltpu.SemaphoreType.DMA,
        pltpu.SemaphoreType.DMA,
    ],
)(_sc_body)


def _mask_body(thr_ref, c_ref, o_ref):
  b = pl.program_id(0)
  v = thr_ref[b, 0].astype(jnp.float32) * (2.0 ** -22)
  c = c_ref[...]
  o_ref[...] = jnp.where(c < v, c, 0.0)


_mask = pl.pallas_call(
    _mask_body,
    grid=(B, NS // RB),
    in_specs=[
        pl.BlockSpec(memory_space=pltpu.SMEM),
        pl.BlockSpec((1, RB, NS), lambda b, j: (b, j, 0)),
    ],
    out_specs=pl.BlockSpec((1, RB, NS), lambda b, j: (b, j, 0)),
    out_shape=jax.ShapeDtypeStruct((B, NS, NS), jnp.float32),
)


def kernel(source, target, cost_matrix):
  cf = cost_matrix.reshape(B, N)
  ithr, _ = _select(cf)
  sparse = _mask(ithr, cost_matrix)
  return source, target, sparse


# speedup vs baseline: 59.3508x; 1.2499x over previous
import functools

import jax
import jax.numpy as jnp
from jax import lax
from jax.experimental import pallas as pl
from jax.experimental.pallas import tpu as pltpu
from jax.experimental.pallas import tpu_sc as plsc

B = 8
NS = 2048
N = NS * NS
K_SEL = int(N * 0.2)
NBINS = 2048
LANES = 16
TPB = 4
SHARD = N // TPB
CHUNK = 8192
NCHUNK = SHARD // CHUNK
VPC = CHUNK // LANES
UNROLL = 4
ROWS = NBINS // LANES
RB = 256

_mesh = plsc.VectorSubcoreMesh(core_axis_name="c", subcore_axis_name="s")


def _scan_threshold(h, rank):
  zeros = jnp.zeros((LANES,), jnp.int32)

  def body(j, carry):
    cum_prev, cnt, maxlt = carry
    v = h[j]
    cs = plsc.cumsum(v) + cum_prev
    lt = cs < rank
    cnt = cnt + lt.astype(jnp.int32)
    maxlt = jnp.where(lt, cs, maxlt)
    return cum_prev + jnp.sum(v), cnt, maxlt

  _, cnt, maxlt = lax.fori_loop(0, ROWS, body, (jnp.int32(0), zeros, zeros))
  return jnp.sum(cnt), jnp.max(maxlt)


def _sc_body(cost_hbm, keys_hbm, thr_hbm, folds_hbm, hist, fold, hmrg,
             buf0, buf1, kbuf0, kbuf1, isem0, isem1):
  cid = lax.axis_index("c")
  sid = lax.axis_index("s")
  bl = sid // 4
  q = sid % 4
  batch = cid * 4 + bl
  base = q * SHARD
  base_k = q * (SHARD // 2)
  lane = lax.iota(jnp.int32, LANES)
  ones = jnp.full((LANES,), 1, jnp.int32)
  zeros = jnp.zeros((LANES,), jnp.int32)

  ibufs = (buf0, buf1)
  isems = (isem0, isem1)

  def in_copy(ci, b):
    return pltpu.make_async_copy(
        cost_hbm.at[batch, pl.ds(base + ci * CHUNK, CHUNK)],
        ibufs[b], isems[b])

  kbufs = (kbuf0, kbuf1)

  def key_copy(ci, b):
    return pltpu.make_async_copy(
        keys_hbm.at[batch, pl.ds(base_k + ci * CHUNK, CHUNK)],
        kbufs[b], isems[b])

  def zero_hist(_i, _):
    for u in range(8):
      hist[pl.ds((_i * 8 + u) * LANES, LANES)] = zeros
    return _

  def fold_hist(j, _):
    rows16 = (j * LANES + lane) * LANES
    accs = [zeros] * 4
    for l in range(LANES):
      accs[l % 4] = accs[l % 4] + plsc.load_gather(hist, [rows16 + l])
    fold[j] = (accs[0] + accs[1]) + (accs[2] + accs[3])
    return _

  def merge_and_scan(rank):
    lax.fori_loop(0, ROWS, fold_hist, 0)
    pltpu.sync_copy(fold, folds_hbm.at[cid, sid])
    plsc.subcore_barrier()

    def add_rows(j, _):
      fold[j] = fold[j] + hmrg[j]
      return _

    for dt in range(1, TPB):
      other = bl * TPB + lax.rem(q + dt, TPB)
      pltpu.sync_copy(folds_hbm.at[cid, other], hmrg)
      lax.fori_loop(0, ROWS, add_rows, 0)
    plsc.subcore_barrier()
    return _scan_threshold(fold, rank)

  def stream_pass(copy_fn, bufs, nchunk, process):
    copy_fn(0, 0).start()

    def pair(g, _):
      ci0 = g * 2
      copy_fn(ci0 + 1, 1).start()
      copy_fn(ci0, 0).wait()
      process(bufs[0])

      @pl.when(ci0 + 2 < nchunk)
      def _():
        copy_fn(ci0 + 2, 0).start()

      copy_fn(ci0 + 1, 1).wait()
      process(bufs[1])
      return 0

    lax.fori_loop(0, nchunk // 2, pair, 0)

  lax.fori_loop(0, NBINS * LANES // (8 * LANES), zero_hist, 0)

  def p1(buf):
    def v1(i, _):
      for u in range(UNROLL):
        w = buf[pl.ds((i * UNROLL + u) * LANES, LANES)]
        blo = ((w >> 5) & 2047) * LANES
        bhi = ((w >> 21) & 2047) * LANES
        plsc.addupdate_scatter(hist, [blo | lane], ones)
        plsc.addupdate_scatter(hist, [bhi | lane], ones)
      return _

    lax.fori_loop(0, VPC // UNROLL, v1, 0)

  stream_pass(key_copy, kbufs, NCHUNK // 2, p1)
  bstar, cum_before = merge_and_scan(K_SEL)
  r = K_SEL - cum_before

  lax.fori_loop(0, NBINS * LANES // (8 * LANES), zero_hist, 0)

  def p2(buf):
    def v2(i, _):
      for u in range(UNROLL):
        c = buf[pl.ds((i * UNROLL + u) * LANES, LANES)]
        m = (c * 4194304.0).astype(jnp.int32)
        inbin = (m >> 11) == bstar
        plsc.addupdate_scatter(hist, [((m & 2047) * LANES) | lane], ones,
                               mask=inbin)
      return _

    lax.fori_loop(0, VPC // UNROLL, v2, 0)

  stream_pass(in_copy, ibufs, NCHUNK, p2)
  sbstar, _cb2 = merge_and_scan(r)

  t = bstar * NBINS + sbstar + 1

  @pl.when(q == 0)
  def _():
    fold[0] = zeros + t
    pltpu.sync_copy(fold.at[0], thr_hbm.at[batch])


_select = functools.partial(
    pl.kernel,
    out_type=(
        jax.ShapeDtypeStruct((B, LANES), jnp.int32),
        jax.ShapeDtypeStruct((2, 16, ROWS, LANES), jnp.int32),
    ),
    mesh=_mesh,
    compiler_params=pltpu.CompilerParams(needs_layout_passes=False),
    scratch_types=[
        pltpu.VMEM((NBINS * LANES,), jnp.int32),
        pltpu.VMEM((ROWS, LANES), jnp.int32),
        pltpu.VMEM((ROWS, LANES), jnp.int32),
        pltpu.VMEM((CHUNK,), jnp.float32),
        pltpu.VMEM((CHUNK,), jnp.float32),
        pltpu.VMEM((CHUNK,), jnp.int32),
        pltpu.VMEM((CHUNK,), jnp.int32),
        pltpu.SemaphoreType.DMA,
        pltpu.SemaphoreType.DMA,
    ],
)(_sc_body)


def _pack_body(c1_ref, c2_ref, o_ref):
  k1 = (c1_ref[...] * 65536.0).astype(jnp.int32)
  k2 = (c2_ref[...] * 65536.0).astype(jnp.int32)
  o_ref[...] = k1 | (k2 << 16)


_pack = pl.pallas_call(
    _pack_body,
    grid=(B, NS // 2 // RB),
    in_specs=[
        pl.BlockSpec((1, RB, NS), lambda b, j: (b, j, 0)),
        pl.BlockSpec((1, RB, NS), lambda b, j: (b, j + NS // 2 // RB, 0)),
    ],
    out_specs=pl.BlockSpec((1, RB, NS), lambda b, j: (b, j, 0)),
    out_shape=jax.ShapeDtypeStruct((B, NS // 2, NS), jnp.int32),
)


def _mask_body(thr_ref, c_ref, o_ref):
  b = pl.program_id(0)
  v = thr_ref[b, 0].astype(jnp.float32) * (2.0 ** -22)
  c = c_ref[...]
  o_ref[...] = jnp.where(c < v, c, 0.0)


_mask = pl.pallas_call(
    _mask_body,
    grid=(B, NS // RB),
    in_specs=[
        pl.BlockSpec(memory_space=pltpu.SMEM),
        pl.BlockSpec((1, RB, NS), lambda b, j: (b, j, 0)),
    ],
    out_specs=pl.BlockSpec((1, RB, NS), lambda b, j: (b, j, 0)),
    out_shape=jax.ShapeDtypeStruct((B, NS, NS), jnp.float32),
)


def kernel(source, target, cost_matrix):
  cf = cost_matrix.reshape(B, N)
  keys = _pack(cost_matrix, cost_matrix).reshape(B, N // 2)
  ithr, _ = _select(cf, keys)
  sparse = _mask(ithr, cost_matrix)
  return source, target, sparse


# speedup vs baseline: 101.1328x; 1.7040x over previous
import functools

import jax
import jax.numpy as jnp
from jax import lax
from jax.experimental import pallas as pl
from jax.experimental.pallas import tpu as pltpu
from jax.experimental.pallas import tpu_sc as plsc

B = 8
NS = 2048
N = NS * NS
K_SEL = int(N * 0.2)
NBINS = 2048
LANES = 16
TPB = 4
SHARD_K = N // 2 // TPB
HALF_K = SHARD_K // 2
SHARD_B = N // 4 // TPB
CHUNK = 8192
NCHUNK1 = SHARD_K // CHUNK
NCHUNK2 = HALF_K // CHUNK
VPC = CHUNK // LANES
UNROLL = 4
ROWS = NBINS // LANES
RB = 128

_mesh = plsc.VectorSubcoreMesh(core_axis_name="c", subcore_axis_name="s")


def _scan_threshold(h, rank):
  zeros = jnp.zeros((LANES,), jnp.int32)

  def body(j, carry):
    cum_prev, cnt, maxlt = carry
    v = h[j]
    cs = plsc.cumsum(v) + cum_prev
    lt = cs < rank
    cnt = cnt + lt.astype(jnp.int32)
    maxlt = jnp.where(lt, cs, maxlt)
    return cum_prev + jnp.sum(v), cnt, maxlt

  _, cnt, maxlt = lax.fori_loop(0, ROWS, body, (jnp.int32(0), zeros, zeros))
  return jnp.sum(cnt), jnp.max(maxlt)


def _sc_body(keys_hbm, bkeys_hbm, thr_hbm, folds_hbm, hist, fold, hmrg,
             kbuf0, kbuf1, hbuf0, hbuf1, bbuf0, bbuf1,
             ksem0, ksem1, hsem0, hsem1, bsem0, bsem1):
  cid = lax.axis_index("c")
  sid = lax.axis_index("s")
  bl = sid // 4
  q = sid % 4
  batch = cid * 4 + bl
  base_k = q * SHARD_K
  base_b = q * SHARD_B
  lane = lax.iota(jnp.int32, LANES)
  ones = jnp.full((LANES,), 1, jnp.int32)
  zeros = jnp.zeros((LANES,), jnp.int32)

  kbufs = (kbuf0, kbuf1)
  hbufs = (hbuf0, hbuf1)
  bbufs = (bbuf0, bbuf1)
  ksems = (ksem0, ksem1)
  hsems = (hsem0, hsem1)
  bsems = (bsem0, bsem1)

  def kcopy(ci, b):
    return pltpu.make_async_copy(
        keys_hbm.at[batch, pl.ds(base_k + ci * CHUNK, CHUNK)],
        kbufs[b], ksems[b])

  def hcopy(ci, b):
    return pltpu.make_async_copy(
        keys_hbm.at[batch, pl.ds(base_k + HALF_K + ci * CHUNK, CHUNK)],
        hbufs[b], hsems[b])

  def bcopy(ci, b):
    return pltpu.make_async_copy(
        bkeys_hbm.at[batch, pl.ds(base_b + ci * CHUNK, CHUNK)],
        bbufs[b], bsems[b])

  def zero_hist(_i, _):
    for u in range(8):
      hist[pl.ds((_i * 8 + u) * LANES, LANES)] = zeros
    return _

  def fold_hist(j, _):
    rows16 = (j * LANES + lane) * LANES
    accs = [zeros] * 4
    for l in range(LANES):
      accs[l % 4] = accs[l % 4] + plsc.load_gather(hist, [rows16 + l])
    fold[j] = (accs[0] + accs[1]) + (accs[2] + accs[3])
    return _

  def merge_and_scan(rank):
    lax.fori_loop(0, ROWS, fold_hist, 0)
    pltpu.sync_copy(fold, folds_hbm.at[cid, sid])
    plsc.subcore_barrier()

    def add_rows(j, _):
      fold[j] = fold[j] + hmrg[j]
      return _

    for dt in range(1, TPB):
      other = bl * TPB + lax.rem(q + dt, TPB)
      pltpu.sync_copy(folds_hbm.at[cid, other], hmrg)
      lax.fori_loop(0, ROWS, add_rows, 0)
    plsc.subcore_barrier()
    return _scan_threshold(fold, rank)

  lax.fori_loop(0, NBINS * LANES // (8 * LANES), zero_hist, 0)

  def p1(buf):
    def v1(i, _):
      for u in range(UNROLL):
        w = buf[pl.ds((i * UNROLL + u) * LANES, LANES)]
        blo = ((w >> 5) & 2047) * LANES
        bhi = ((w >> 21) & 2047) * LANES
        plsc.addupdate_scatter(hist, [blo | lane], ones)
        plsc.addupdate_scatter(hist, [bhi | lane], ones)
      return _

    lax.fori_loop(0, VPC // UNROLL, v1, 0)

  kcopy(0, 0).start()

  def pair1(g, _):
    ci0 = g * 2
    kcopy(ci0 + 1, 1).start()
    kcopy(ci0, 0).wait()
    p1(kbufs[0])

    @pl.when(ci0 + 2 < NCHUNK1)
    def _():
      kcopy(ci0 + 2, 0).start()

    kcopy(ci0 + 1, 1).wait()
    p1(kbufs[1])
    return 0

  lax.fori_loop(0, NCHUNK1 // 2, pair1, 0)
  bstar, cum_before = merge_and_scan(K_SEL)
  r = K_SEL - cum_before

  lax.fori_loop(0, NBINS * LANES // (8 * LANES), zero_hist, 0)

  def p2(kb, hb, bb):
    def v2(i, _):
      for u in range(UNROLL):
        s = pl.ds((i * UNROLL + u) * LANES, LANES)
        wlo = kb[s]
        whi = hb[s]
        wb = bb[s]
        for w, hsh, bsh in ((wlo, 0, 0), (wlo, 16, 6),
                            (whi, 0, 12), (whi, 16, 18)):
          coarse = (w >> (5 + hsh)) & 2047
          idx = ((((w >> hsh) & 31) << 10)
                 | (((wb >> bsh) & 63) << 4) | lane)
          plsc.addupdate_scatter(hist, [idx], ones, mask=coarse == bstar)
      return _

    lax.fori_loop(0, VPC // UNROLL, v2, 0)

  kcopy(0, 0).start()
  hcopy(0, 0).start()
  bcopy(0, 0).start()

  def pair2(g, _):
    ci0 = g * 2
    kcopy(ci0 + 1, 1).start()
    hcopy(ci0 + 1, 1).start()
    bcopy(ci0 + 1, 1).start()
    kcopy(ci0, 0).wait()
    hcopy(ci0, 0).wait()
    bcopy(ci0, 0).wait()
    p2(kbufs[0], hbufs[0], bbufs[0])

    @pl.when(ci0 + 2 < NCHUNK2)
    def _():
      kcopy(ci0 + 2, 0).start()
      hcopy(ci0 + 2, 0).start()
      bcopy(ci0 + 2, 0).start()

    kcopy(ci0 + 1, 1).wait()
    hcopy(ci0 + 1, 1).wait()
    bcopy(ci0 + 1, 1).wait()
    p2(kbufs[1], hbufs[1], bbufs[1])
    return 0

  lax.fori_loop(0, NCHUNK2 // 2, pair2, 0)
  sbstar, _cb2 = merge_and_scan(r)

  t = bstar * NBINS + sbstar + 1

  @pl.when(q == 0)
  def _():
    fold[0] = zeros + t
    pltpu.sync_copy(fold.at[0], thr_hbm.at[batch])


_select = functools.partial(
    pl.kernel,
    out_type=(
        jax.ShapeDtypeStruct((B, LANES), jnp.int32),
        jax.ShapeDtypeStruct((2, 16, ROWS, LANES), jnp.int32),
    ),
    mesh=_mesh,
    compiler_params=pltpu.CompilerParams(needs_layout_passes=False),
    scratch_types=[
        pltpu.VMEM((NBINS * LANES,), jnp.int32),
        pltpu.VMEM((ROWS, LANES), jnp.int32),
        pltpu.VMEM((ROWS, LANES), jnp.int32),
        pltpu.VMEM((CHUNK,), jnp.int32),
        pltpu.VMEM((CHUNK,), jnp.int32),
        pltpu.VMEM((CHUNK,), jnp.int32),
        pltpu.VMEM((CHUNK,), jnp.int32),
        pltpu.VMEM((CHUNK,), jnp.int32),
        pltpu.VMEM((CHUNK,), jnp.int32),
        pltpu.SemaphoreType.DMA,
        pltpu.SemaphoreType.DMA,
        pltpu.SemaphoreType.DMA,
        pltpu.SemaphoreType.DMA,
        pltpu.SemaphoreType.DMA,
        pltpu.SemaphoreType.DMA,
    ],
)(_sc_body)


def _pack_body(c0_ref, c1_ref, c2_ref, c3_ref, keys_ref, bk_ref):
  m0 = (c0_ref[...] * 4194304.0).astype(jnp.int32)
  m1 = (c1_ref[...] * 4194304.0).astype(jnp.int32)
  m2 = (c2_ref[...] * 4194304.0).astype(jnp.int32)
  m3 = (c3_ref[...] * 4194304.0).astype(jnp.int32)
  a01 = (m0 >> 6) | ((m1 >> 6) << 16)
  a23 = (m2 >> 6) | ((m3 >> 6) << 16)
  keys_ref[...] = jnp.concatenate([a01, a23], axis=1)
  bk_ref[...] = ((m0 & 63) | ((m1 & 63) << 6)
                 | ((m2 & 63) << 12) | ((m3 & 63) << 18))


_pack = pl.pallas_call(
    _pack_body,
    grid=(B, TPB),
    in_specs=[
        pl.BlockSpec((1, RB, NS), lambda b, j: (b, 2 * j, 0)),
        pl.BlockSpec((1, RB, NS), lambda b, j: (b, 2 * j + 8, 0)),
        pl.BlockSpec((1, RB, NS), lambda b, j: (b, 2 * j + 1, 0)),
        pl.BlockSpec((1, RB, NS), lambda b, j: (b, 2 * j + 9, 0)),
    ],
    out_specs=[
        pl.BlockSpec((1, 2 * RB, NS), lambda b, j: (b, j, 0)),
        pl.BlockSpec((1, RB, NS), lambda b, j: (b, j, 0)),
    ],
    out_shape=[
        jax.ShapeDtypeStruct((B, NS // 2, NS), jnp.int32),
        jax.ShapeDtypeStruct((B, NS // 4, NS), jnp.int32),
    ],
)


def _mask_body(thr_ref, c_ref, o_ref):
  b = pl.program_id(0)
  v = thr_ref[b, 0].astype(jnp.float32) * (2.0 ** -22)
  c = c_ref[...]
  o_ref[...] = jnp.where(c < v, c, 0.0)


_mask = pl.pallas_call(
    _mask_body,
    grid=(B, NS // (2 * RB)),
    in_specs=[
        pl.BlockSpec(memory_space=pltpu.SMEM),
        pl.BlockSpec((1, 2 * RB, NS), lambda b, j: (b, j, 0)),
    ],
    out_specs=pl.BlockSpec((1, 2 * RB, NS), lambda b, j: (b, j, 0)),
    out_shape=jax.ShapeDtypeStruct((B, NS, NS), jnp.float32),
)


def kernel(source, target, cost_matrix):
  keys, bkeys = _pack(cost_matrix, cost_matrix, cost_matrix, cost_matrix)
  ithr, _ = _select(keys.reshape(B, N // 2), bkeys.reshape(B, N // 4))
  sparse = _mask(ithr, cost_matrix)
  return source, target, sparse
